# Initial kernel scaffold; baseline (speedup 1.0000x reference)
import jax
import jax.numpy as jnp
from jax.experimental import pallas as pl


def _side_info_block(tab_t_ref, out_ref):
    K = out_ref.shape[2]
    L = out_ref.shape[3]
    C_TIME = 128
    c = jax.lax.broadcasted_iota(jnp.float32, (C_TIME, L), 0)
    l = jax.lax.broadcasted_iota(jnp.float32, (C_TIME, L), 1)
    c_even = c - jnp.mod(c, 2.0)
    ln10000 = 9.210340371976184
    div = jnp.exp(c_even * (-ln10000 / 128.0))
    angle = l * div
    is_even = jnp.mod(c, 2.0) < 0.5
    pe = jnp.where(is_even, jnp.sin(angle), jnp.cos(angle))
    out_ref[0, 0:C_TIME, :, :] = jnp.broadcast_to(pe[:, None, :], (C_TIME, K, L))
    tab = tab_t_ref[...]
    out_ref[0, C_TIME:144, :, :] = jnp.broadcast_to(tab[:, :, None], (16, K, L))


def _side_info(tab_t, B, K, L):
    return pl.pallas_call(
        _side_info_block,
        grid=(B,),
        in_specs=[pl.BlockSpec((16, K), lambda b: (0, 0))],
        out_specs=pl.BlockSpec((1, 144, K, L), lambda b: (b, 0, 0, 0)),
        out_shape=jax.ShapeDtypeStruct((B, 144, K, L), jnp.float32),
    )(tab_t)


def kernel(cond_mask, table):
    B, _, K, L = cond_mask.shape
    tab_t = table.T
    return _side_info(tab_t, B, K, L)


# speedup vs baseline: 3.0188x; 3.0188x over previous
import jax
import jax.numpy as jnp
from jax.experimental import pallas as pl


def _side_info_block(tab_t_ref, out_ref):
    K = out_ref.shape[2]
    L = out_ref.shape[3]
    C_TIME = 128
    ci = jax.lax.broadcasted_iota(jnp.int32, (C_TIME, L), 0)
    li = jax.lax.broadcasted_iota(jnp.int32, (C_TIME, L), 1)
    c_rem = ci - (ci // 2) * 2
    c_even = (ci - c_rem).astype(jnp.float32)
    ln10000 = 9.210340371976184
    div = jnp.exp(c_even * (-ln10000 / 128.0))
    angle = li.astype(jnp.float32) * div
    is_even = c_rem == 0
    pe = jnp.where(is_even, jnp.sin(angle), jnp.cos(angle))
    out_ref[0, 0:C_TIME, :, :] = jnp.broadcast_to(pe[:, None, :], (C_TIME, K, L))
    tab = tab_t_ref[...]
    out_ref[0, C_TIME:144, :, :] = jnp.broadcast_to(tab[:, :, None], (16, K, L))


def _side_info(tab_t, B, K, L):
    return pl.pallas_call(
        _side_info_block,
        grid=(B,),
        in_specs=[pl.BlockSpec((16, K), lambda b: (0, 0))],
        out_specs=pl.BlockSpec((1, 144, K, L), lambda b: (b, 0, 0, 0)),
        out_shape=jax.ShapeDtypeStruct((B, 144, K, L), jnp.float32),
    )(tab_t)


def kernel(cond_mask, table):
    B, _, K, L = cond_mask.shape
    tab_t = table.T
    return _side_info(tab_t, B, K, L)
